# baseline (speedup 1.0000x reference)
import jax
import jax.numpy as jnp
from jax.experimental import pallas as pl
from jax.experimental.pallas import tpu as pltpu

_SMOOTHING = 0.2
_CONFIDENCE = 1.0 - _SMOOTHING
_ROWS_PER_BLOCK = 32


def _loss_block_kernel(pred_ref, tgt_ref, out_ref):
    x = pred_ref[...]
    rb, c = x.shape
    t = tgt_ref[0]
    cols = jax.lax.broadcasted_iota(jnp.int32, (rb, c), 1)
    center = cols == t
    window = (cols >= t - 1) & (cols <= t + 1)
    a = jnp.sum(jnp.where(center, x, 0.0), axis=-1, keepdims=True)
    w = jnp.sum(jnp.where(window, x, 0.0), axis=-1, keepdims=True)
    m = jnp.max(x, axis=-1, keepdims=True)
    s = jnp.sum(jnp.exp(x - m), axis=-1, keepdims=True)
    lse = m + jnp.log(s)
    edge = (t == 0) | (t == c - 1)
    sr = jnp.where(edge, _SMOOTHING, 0.5 * _SMOOTHING)
    g = (_CONFIDENCE - sr) * a + sr * w
    out_ref[0] = lse - g


def kernel(pred, target):
    b, c = pred.shape
    rb = _ROWS_PER_BLOCK
    nb = b // rb
    tgt = target.astype(jnp.int32).reshape(nb, rb, 1)
    losses = pl.pallas_call(
        _loss_block_kernel,
        grid=(nb,),
        in_specs=[
            pl.BlockSpec((rb, c), lambda i: (i, 0)),
            pl.BlockSpec((1, rb, 1), lambda i: (i, 0, 0)),
        ],
        out_specs=pl.BlockSpec((1, rb, 1), lambda i: (i, 0, 0)),
        out_shape=jax.ShapeDtypeStruct((nb, rb, 1), jnp.float32),
        compiler_params=pltpu.CompilerParams(
            dimension_semantics=("parallel",),
            vmem_limit_bytes=64 * 1024 * 1024,
        ),
    )(pred, tgt)
    return jnp.mean(losses.reshape(b))


# speedup vs baseline: 1.1287x; 1.1287x over previous
import jax
import jax.numpy as jnp
from jax.experimental import pallas as pl
from jax.experimental.pallas import tpu as pltpu

_SMOOTHING = 0.2
_CONFIDENCE = 1.0 - _SMOOTHING
_ROWS_PER_BLOCK = 32


def _loss_block_kernel(tgt_smem, pred_ref, out_ref, wx_scratch):
    rb, c = pred_ref.shape
    i = pl.program_id(0)
    last_base = ((c - 1) // 128) * 128
    lane = jax.lax.broadcasted_iota(jnp.int32, (1, 128), 1)

    for r in range(rb):
        t = tgt_smem[i * rb + r]
        lo = t - 1
        hi = jnp.minimum(t + 1, c - 1)
        b0 = jax.lax.shift_left(jax.lax.shift_right_logical(jnp.maximum(lo, 0), 7), 7)
        b1 = jnp.where(b0 + 128 <= last_base, b0 + 128, b0)
        edge = jnp.logical_or(t == 0, t == c - 1)
        s = jnp.where(edge, _SMOOTHING, 0.5 * _SMOOTHING)

        for k, b in enumerate((b0, b1)):
            b = pl.multiple_of(b, 128)
            chunk = pred_ref[r : r + 1, pl.ds(b, 128)]
            col = lane + b
            in_win = jnp.logical_and(col >= lo, col <= hi)
            w = jnp.where(col == t, _CONFIDENCE, jnp.where(in_win, s, 0.0))
            if k == 1:
                w = jnp.where(col >= b0 + 128, w, 0.0)
            wx_scratch[r : r + 1, 128 * k : 128 * (k + 1)] = w * chunk

    x = pred_ref[...]
    m = jnp.max(x, axis=-1, keepdims=True)
    e = jnp.sum(jnp.exp(x - m), axis=-1, keepdims=True)
    lse = m + jnp.log(e)
    g = jnp.sum(wx_scratch[...], axis=-1, keepdims=True)
    out_ref[0] = lse - g


def kernel(pred, target):
    b, c = pred.shape
    rb = _ROWS_PER_BLOCK
    nb = b // rb
    losses = pl.pallas_call(
        _loss_block_kernel,
        grid_spec=pltpu.PrefetchScalarGridSpec(
            num_scalar_prefetch=1,
            grid=(nb,),
            in_specs=[pl.BlockSpec((rb, c), lambda i, tgt: (i, 0))],
            out_specs=pl.BlockSpec((1, rb, 1), lambda i, tgt: (i, 0, 0)),
            scratch_shapes=[pltpu.VMEM((rb, 256), jnp.float32)],
        ),
        out_shape=jax.ShapeDtypeStruct((nb, rb, 1), jnp.float32),
        compiler_params=pltpu.CompilerParams(
            dimension_semantics=("parallel",),
            vmem_limit_bytes=64 * 1024 * 1024,
        ),
    )(target.astype(jnp.int32), pred)
    return jnp.mean(losses.reshape(b))


# speedup vs baseline: 1.4766x; 1.3082x over previous
import jax
import jax.numpy as jnp
from jax.experimental import pallas as pl
from jax.experimental.pallas import tpu as pltpu

_SMOOTHING = 0.2
_CONFIDENCE = 1.0 - _SMOOTHING
_ROW_CHUNK = 1024
_NEG = -1e30


def _loss_block_kernel(predt_ref, tgt_ref, out_ref, m_run, s_run, g_run, nc, c):
    i = pl.program_id(1)
    rk, nb = predt_ref.shape

    @pl.when(i == 0)
    def _init():
        m_run[...] = jnp.full((1, nb), _NEG, jnp.float32)
        s_run[...] = jnp.zeros((1, nb), jnp.float32)
        g_run[...] = jnp.zeros((1, nb), jnp.float32)

    x = predt_ref[...]
    t = tgt_ref[0]
    rows = jax.lax.broadcasted_iota(jnp.int32, (rk, nb), 0) + i * rk
    valid = rows < c
    xv = jnp.where(valid, x, _NEG)

    m_old = m_run[...]
    m_new = jnp.maximum(m_old, jnp.max(xv, axis=0, keepdims=True))
    bs = jnp.sum(jnp.exp(xv - m_new), axis=0, keepdims=True)
    s_run[...] = s_run[...] * jnp.exp(m_old - m_new) + bs
    m_run[...] = m_new

    edge = jnp.logical_or(t == 0, t == c - 1)
    s = jnp.where(edge, _SMOOTHING, 0.5 * _SMOOTHING)
    in_win = jnp.logical_and(
        jnp.logical_and(rows >= t - 1, rows <= t + 1), valid
    )
    w = jnp.where(rows == t, _CONFIDENCE, jnp.where(in_win, s, 0.0))
    g_run[...] = g_run[...] + jnp.sum(w * xv, axis=0, keepdims=True)

    @pl.when(i == nc - 1)
    def _fini():
        out_ref[0] = m_run[...] + jnp.log(s_run[...]) - g_run[...]


def kernel(pred, target):
    b, c = pred.shape
    predt = pred.T
    rk = _ROW_CHUNK
    nc = pl.cdiv(c, rk)
    nbh = b // 2
    tgt = target.astype(jnp.int32).reshape(2, 1, nbh)

    import functools

    body = functools.partial(_loss_block_kernel, nc=nc, c=c)
    losses = pl.pallas_call(
        body,
        grid=(2, nc),
        in_specs=[
            pl.BlockSpec((rk, nbh), lambda j, i: (i, j)),
            pl.BlockSpec((1, 1, nbh), lambda j, i: (j, 0, 0)),
        ],
        out_specs=pl.BlockSpec((1, 1, nbh), lambda j, i: (j, 0, 0)),
        out_shape=jax.ShapeDtypeStruct((2, 1, nbh), jnp.float32),
        scratch_shapes=[
            pltpu.VMEM((1, nbh), jnp.float32),
            pltpu.VMEM((1, nbh), jnp.float32),
            pltpu.VMEM((1, nbh), jnp.float32),
        ],
        compiler_params=pltpu.CompilerParams(
            dimension_semantics=("parallel", "arbitrary"),
            vmem_limit_bytes=64 * 1024 * 1024,
        ),
    )(predt, tgt)
    return jnp.mean(losses.reshape(b))


# speedup vs baseline: 2.9897x; 2.0248x over previous
import functools

import jax
import jax.numpy as jnp
from jax.experimental import pallas as pl
from jax.experimental.pallas import tpu as pltpu

_SMOOTHING = 0.2
_CONFIDENCE = 1.0 - _SMOOTHING
_ROW_CHUNK = 1024
_NEG = -1e30


def _loss_block_kernel(predt_ref, tgt_ref, out_ref, m8, s8, a8, w8, nc, c):
    i = pl.program_id(1)
    rk, nb = predt_ref.shape
    nt = rk // 8

    @pl.when(i == 0)
    def _init():
        m8[...] = jnp.full((8, nb), _NEG, jnp.float32)
        s8[...] = jnp.zeros((8, nb), jnp.float32)
        a8[...] = jnp.zeros((8, nb), jnp.float32)
        w8[...] = jnp.zeros((8, nb), jnp.float32)

    t = tgt_ref[0]

    def _step(mask_tail):
        x3 = predt_ref[...].reshape(nt, 8, nb)
        rows = (
            jax.lax.broadcasted_iota(jnp.int32, (nt, 8, nb), 0) * 8
            + jax.lax.broadcasted_iota(jnp.int32, (nt, 8, nb), 1)
            + i * rk
        )
        if mask_tail:
            xm = jnp.where(rows < c, x3, _NEG)
        else:
            xm = x3
        m_new = jnp.maximum(m8[...], jnp.max(xm, axis=0))
        s8[...] = s8[...] * jnp.exp(m8[...] - m_new) + jnp.sum(
            jnp.exp(xm - m_new[None]), axis=0
        )
        m8[...] = m_new
        center = rows == t[None]
        win = jnp.logical_and(rows >= t[None] - 1, rows <= t[None] + 1)
        if mask_tail:
            win = jnp.logical_and(win, rows < c)
        a8[...] = a8[...] + jnp.sum(jnp.where(center, x3, 0.0), axis=0)
        w8[...] = w8[...] + jnp.sum(jnp.where(win, x3, 0.0), axis=0)

    @pl.when(i < nc - 1)
    def _fast():
        _step(False)

    @pl.when(i == nc - 1)
    def _tail():
        _step(True)

        m_f = jnp.max(m8[...], axis=0, keepdims=True)
        s_f = jnp.sum(s8[...] * jnp.exp(m8[...] - m_f), axis=0, keepdims=True)
        a = jnp.sum(a8[...], axis=0, keepdims=True)
        w = jnp.sum(w8[...], axis=0, keepdims=True)
        edge = jnp.logical_or(t == 0, t == c - 1)
        s = jnp.where(edge, _SMOOTHING, 0.5 * _SMOOTHING)
        g = (_CONFIDENCE - s) * a + s * w
        out_ref[0] = m_f + jnp.log(s_f) - g


def kernel(pred, target):
    b, c = pred.shape
    predt = pred.T
    rk = _ROW_CHUNK
    nc = pl.cdiv(c, rk)
    nbh = b // 2
    tgt = target.astype(jnp.int32).reshape(2, 1, nbh)

    body = functools.partial(_loss_block_kernel, nc=nc, c=c)
    losses = pl.pallas_call(
        body,
        grid=(2, nc),
        in_specs=[
            pl.BlockSpec((rk, nbh), lambda j, i: (i, j)),
            pl.BlockSpec((1, 1, nbh), lambda j, i: (j, 0, 0)),
        ],
        out_specs=pl.BlockSpec((1, 1, nbh), lambda j, i: (j, 0, 0)),
        out_shape=jax.ShapeDtypeStruct((2, 1, nbh), jnp.float32),
        scratch_shapes=[
            pltpu.VMEM((8, nbh), jnp.float32),
            pltpu.VMEM((8, nbh), jnp.float32),
            pltpu.VMEM((8, nbh), jnp.float32),
            pltpu.VMEM((8, nbh), jnp.float32),
        ],
        compiler_params=pltpu.CompilerParams(
            dimension_semantics=("parallel", "arbitrary"),
            vmem_limit_bytes=64 * 1024 * 1024,
        ),
    )(predt, tgt)
    return jnp.mean(losses.reshape(b))
